# baseline (speedup 1.0000x reference)
import functools

import jax
import jax.numpy as jnp
from jax.experimental import pallas as pl


def _body(x_ref, m_ref, o_ref, *, hw, r, s):
    ch = 8
    k_steps = s // ch
    zero = jnp.zeros((r, ch, 128), jnp.float32)
    zrow = jnp.zeros((ch, 128), jnp.float32)

    def stats_body(k, carry):
        sa, qa, sf, qf, nf = carry
        off = pl.multiple_of(k * ch, ch)
        m = m_ref[0, pl.ds(off, ch), :]
        fgm = m >= 0.5
        xc = x_ref[:, pl.ds(off, ch), :]
        sq = xc * xc
        xm = jnp.where(fgm, xc, 0.0)
        qm = jnp.where(fgm, sq, 0.0)
        nf = nf + jnp.where(fgm, 1.0, 0.0)
        return sa + xc, qa + sq, sf + xm, qf + qm, nf

    sa, qa, sf, qf, nf = jax.lax.fori_loop(
        0, k_steps, stats_body, (zero, zero, zero, zero, zrow))

    n_fg = jnp.sum(nf)
    n_bg = hw - n_fg
    s_all = jnp.sum(sa, axis=(1, 2))
    s_fg = jnp.sum(sf, axis=(1, 2))
    q_all = jnp.sum(qa, axis=(1, 2))
    q_fg = jnp.sum(qf, axis=(1, 2))

    mu_fg = s_fg / n_fg
    mu_bg = (s_all - s_fg) / n_bg
    var_fg = (q_fg - n_fg * mu_fg * mu_fg) / (n_fg - 1.0)
    var_bg = ((q_all - q_fg) - n_bg * mu_bg * mu_bg) / (n_bg - 1.0)
    scale = jnp.sqrt(var_fg) / (jnp.sqrt(var_bg) + 1e-8)
    shift = (mu_fg - scale * mu_bg)[:, None, None]
    scale = scale[:, None, None]

    def write_body(k, _):
        off = pl.multiple_of(k * ch, ch)
        m = m_ref[0, pl.ds(off, ch), :]
        fgm = m >= 0.5
        xc = x_ref[:, pl.ds(off, ch), :]
        y = xc * scale + shift
        o_ref[:, pl.ds(off, ch), :] = jnp.where(fgm, xc, y)
        return 0

    jax.lax.fori_loop(0, k_steps, write_body, 0)


def kernel(x, mask):
    b, c, h, w = x.shape
    hw = h * w
    s = hw // 128
    x3 = x.reshape(b * c, s, 128)
    m3 = mask.reshape(b, s, 128)

    r = 8 if c % 8 == 0 else 1
    grid = (b * c) // r
    rows_per_b = c // r

    out = pl.pallas_call(
        functools.partial(_body, hw=float(hw), r=r, s=s),
        grid=(grid,),
        in_specs=[
            pl.BlockSpec((r, s, 128), lambda i: (i, 0, 0)),
            pl.BlockSpec((1, s, 128), lambda i: (i // rows_per_b, 0, 0)),
        ],
        out_specs=pl.BlockSpec((r, s, 128), lambda i: (i, 0, 0)),
        out_shape=jax.ShapeDtypeStruct((b * c, s, 128), x.dtype),
    )(x3, m3)
    return out.reshape(b, c, h, w)


# speedup vs baseline: 2.7032x; 2.7032x over previous
import functools

import jax
import jax.numpy as jnp
from jax.experimental import pallas as pl


def _body(x_ref, m_ref, o_ref, *, hw, r, s, w):
    ch = 8
    k_steps = s // ch
    zero = jnp.zeros((r, ch, w), jnp.float32)
    zrow = jnp.zeros((ch, w), jnp.float32)

    def stats_body(k, carry):
        sa, qa, sf, qf, nf = carry
        off = pl.multiple_of(k * ch, ch)
        m = m_ref[0, pl.ds(off, ch), :]
        fgm = m >= 0.5
        xc = x_ref[:, pl.ds(off, ch), :]
        sq = xc * xc
        xm = jnp.where(fgm, xc, 0.0)
        qm = jnp.where(fgm, sq, 0.0)
        nf = nf + jnp.where(fgm, 1.0, 0.0)
        return sa + xc, qa + sq, sf + xm, qf + qm, nf

    sa, qa, sf, qf, nf = jax.lax.fori_loop(
        0, k_steps, stats_body, (zero, zero, zero, zero, zrow))

    n_fg = jnp.sum(nf)
    n_bg = hw - n_fg
    s_all = jnp.sum(sa, axis=(1, 2))
    s_fg = jnp.sum(sf, axis=(1, 2))
    q_all = jnp.sum(qa, axis=(1, 2))
    q_fg = jnp.sum(qf, axis=(1, 2))

    mu_fg = s_fg / n_fg
    mu_bg = (s_all - s_fg) / n_bg
    var_fg = (q_fg - n_fg * mu_fg * mu_fg) / (n_fg - 1.0)
    var_bg = ((q_all - q_fg) - n_bg * mu_bg * mu_bg) / (n_bg - 1.0)
    scale = jnp.sqrt(var_fg) / (jnp.sqrt(var_bg) + 1e-8)
    shift = (mu_fg - scale * mu_bg)[:, None, None]
    scale = scale[:, None, None]

    def write_body(k, _):
        off = pl.multiple_of(k * ch, ch)
        m = m_ref[0, pl.ds(off, ch), :]
        fgm = m >= 0.5
        xc = x_ref[:, pl.ds(off, ch), :]
        y = xc * scale + shift
        o_ref[:, pl.ds(off, ch), :] = jnp.where(fgm, xc, y)
        return 0

    jax.lax.fori_loop(0, k_steps, write_body, 0)


def kernel(x, mask):
    b, c, h, w = x.shape
    hw = h * w
    x3 = x.reshape(b * c, h, w)
    m3 = mask.reshape(b, h, w)

    r = 8 if c % 8 == 0 else 1
    grid = (b * c) // r
    rows_per_b = c // r

    out = pl.pallas_call(
        functools.partial(_body, hw=float(hw), r=r, s=h, w=w),
        grid=(grid,),
        in_specs=[
            pl.BlockSpec((r, h, w), lambda i: (i, 0, 0)),
            pl.BlockSpec((1, h, w), lambda i: (i // rows_per_b, 0, 0)),
        ],
        out_specs=pl.BlockSpec((r, h, w), lambda i: (i, 0, 0)),
        out_shape=jax.ShapeDtypeStruct((b * c, h, w), x.dtype),
    )(x3, m3)
    return out.reshape(b, c, h, w)
